# baseline (speedup 1.0000x reference)
import functools

import jax
import jax.numpy as jnp
from jax import lax
from jax.experimental import pallas as pl
from jax.experimental.pallas import tpu as pltpu
from jax.experimental.pallas import tpu_sc as plsc

_N = 1024
_H = 125
_HC = 25
_SIZES = (61, 31, 15)
_SCALES = (4, 2, 1)
_PADS = (16, 8, 4)
_LANES = 16


def _compute_plane(dk, ok, size, delta, cols):
    def body_u(u, carry):
        s_row = delta + u
        rvalid = jnp.logical_and(s_row >= 0, s_row < size)
        rid = jnp.clip(s_row, 0, size - 1)
        ridv = jnp.broadcast_to(rid, (_LANES,))
        for j, (ccl, cval, oc, smask) in enumerate(cols):
            g = plsc.load_gather(dk, [ridv, ccl])
            val = jnp.where(jnp.logical_and(cval, rvalid), g, jnp.float32(0))
            if smask is None:
                ok[u, pl.ds(_LANES * j, _LANES)] = val
            else:
                uv = jnp.broadcast_to(u, (_LANES,))
                plsc.store_scatter(ok, [uv, oc], val, mask=smask)
        return carry

    lax.fori_loop(0, size, body_u, jnp.int32(0))


def _body(ff, co3, anch, p0, p1, p2, p3,
          av, d0a, d1a, d2a, d0b, d1b, d2b,
          o0a, o1a, o2a, o0b, o1b, o2b, p3buf, p3o,
          sem_ia, sem_ib, sem_oa, sem_ob, sem_m, *, nsub, per_worker):
    outs = (p0, p1, p2)
    din_a = (d0a, d1a, d2a)
    din_b = (d0b, d1b, d2b)
    ok_a = (o0a, o1a, o2a)
    ok_b = (o0b, o1b, o2b)

    wid = lax.axis_index("c") * nsub + lax.axis_index("s")
    base = wid * per_worker

    pltpu.sync_copy(anch, av)
    r = jnp.max(av[pl.ds(0, _LANES)])
    c = jnp.max(av[pl.ds(_LANES, _LANES)])

    lane = lax.iota(jnp.int32, _LANES)

    cl_r, deltas, cols = [], [], []
    for k in range(3):
        size, scale, pad = _SIZES[k], _SCALES[k], _PADS[k]
        r0 = scale * r - pad
        clr = jnp.clip(r0, 0, _H - size)
        cl_r.append(clr)
        deltas.append(r0 - clr)
        c0 = scale * c - pad
        chunks = []
        for j in range((size + _LANES - 1) // _LANES):
            oc = _LANES * j + lane
            cid = c0 + oc
            ccl = jnp.clip(cid, 0, _H - 1)
            cval = jnp.logical_and(cid >= 0, cid < _H)
            smask = None if _LANES * (j + 1) <= size else oc < size
            chunks.append((ccl, cval, oc, smask))
        cols.append(chunks)

    cin = pltpu.async_copy(
        co3.at[pl.ds(base, per_worker), pl.ds(r, 1), :], p3buf, sem_m)

    def issue_in(i, bufs, sem):
        return [pltpu.async_copy(
            ff.at[i, pl.ds(cl_r[k], _SIZES[k]), :], bufs[k], sem)
            for k in range(3)]

    def issue_out(i, bufs, sem):
        return [pltpu.async_copy(bufs[k], outs[k].at[i], sem)
                for k in range(3)]

    def compute_all(dbufs, obufs):
        for k in range(3):
            _compute_plane(dbufs[k], obufs[k], _SIZES[k], deltas[k], cols[k])

    def step(g, carry):
        i0 = base + 2 * g
        i1 = i0 + 1
        ia = issue_in(i0, din_a, sem_ia)
        ib = issue_in(i1, din_b, sem_ib)
        for cp in ia:
            cp.wait()
        compute_all(din_a, ok_a)
        oa = issue_out(i0, ok_a, sem_oa)
        for cp in ib:
            cp.wait()
        compute_all(din_b, ok_b)
        ob = issue_out(i1, ok_b, sem_ob)
        for cp in oa:
            cp.wait()
        for cp in ob:
            cp.wait()
        return carry

    lax.fori_loop(0, per_worker // 2, step, jnp.int32(0))

    cin.wait()
    zero = jnp.broadcast_to(jnp.int32(0), (_LANES,))
    cv = jnp.broadcast_to(c, (_LANES,))
    for j in range(per_worker // _LANES):
        rows = _LANES * j + lane
        p3o[pl.ds(_LANES * j, _LANES)] = plsc.load_gather(
            p3buf, [rows, zero, cv])
    pltpu.sync_copy(p3o, p3.at[pl.ds(base, per_worker)])


def kernel(full_feature, corr_feature, anchor):
    B, C, H, W = full_feature.shape
    ff = full_feature.reshape(B * C, H, W)
    co3 = corr_feature.reshape(B * C, _HC, _HC)
    a32 = anchor.astype(jnp.int32)
    anch = jnp.concatenate([
        jnp.broadcast_to(a32[0], (_LANES,)),
        jnp.broadcast_to(a32[1], (_LANES,)),
    ])
    f32 = jnp.float32

    mesh = plsc.VectorSubcoreMesh(core_axis_name="c", subcore_axis_name="s")
    nw = mesh.num_cores * mesh.num_subcores
    per_worker = _N // nw

    dbuf = [pltpu.VMEM((_SIZES[k], _H), f32) for k in range(3)]
    obuf = [pltpu.VMEM((_SIZES[k], _SIZES[k]), f32) for k in range(3)]

    run = pl.kernel(
        functools.partial(_body, nsub=mesh.num_subcores,
                          per_worker=per_worker),
        out_type=(
            jax.ShapeDtypeStruct((_N, 61, 61), f32),
            jax.ShapeDtypeStruct((_N, 31, 31), f32),
            jax.ShapeDtypeStruct((_N, 15, 15), f32),
            jax.ShapeDtypeStruct((_N,), f32),
        ),
        mesh=mesh,
        compiler_params=pltpu.CompilerParams(use_tc_tiling_on_sc=False,
                                             needs_layout_passes=False),
        scratch_types=[
            pltpu.VMEM((2 * _LANES,), jnp.int32),
            *dbuf, *dbuf,
            *obuf, *obuf,
            pltpu.VMEM((per_worker, 1, _HC), f32),
            pltpu.VMEM((per_worker,), f32),
            pltpu.SemaphoreType.DMA,
            pltpu.SemaphoreType.DMA,
            pltpu.SemaphoreType.DMA,
            pltpu.SemaphoreType.DMA,
            pltpu.SemaphoreType.DMA,
        ],
    )
    p0, p1, p2, p3 = run(ff, co3, anch)
    return (p0.reshape(B, C, 61, 61), p1.reshape(B, C, 31, 31),
            p2.reshape(B, C, 15, 15), p3.reshape(B, C, 1, 1))


# speedup vs baseline: 7.0769x; 7.0769x over previous
import functools

import jax
import jax.numpy as jnp
from jax import lax
from jax.experimental import pallas as pl
from jax.experimental.pallas import tpu as pltpu
from jax.experimental.pallas import tpu_sc as plsc

_H = 125
_HC = 25
_LANES = 16
_U = 31
_SIZES = (61, 31, 15)
_SCALES = (4, 2, 1)
_PADS = (16, 8, 4)
_TYPES = ((0, 0, 31), (0, 31, 30), (1, 0, 31), (2, 0, 15))
_STARTS = (0, 61, 122, 153)
_NUNITS = 168


def _to_rows(x):
    B, C, H, W = x.shape
    y = x.reshape(B, 2, 128, H, W).transpose(3, 4, 1, 0, 2)
    return y.reshape(H * W, 8, 128)


def _from_rows(y, H, W):
    z = y.reshape(H, W, 2, 4, 128).transpose(3, 2, 4, 0, 1)
    return z.reshape(4, 256, H, W)


def _body(ffr, cor, anch, zin, q0, q1, q2, q3,
          av, bufa, bufb, zer, idxa, idxb,
          sem_ga, sem_gb, sem_o, sem_m, *, nsub, nworkers):
    outs = (q0, q1, q2)
    bufs = (bufa, bufb)
    idxs = (idxa, idxb)
    gsems = (sem_ga, sem_gb)

    wid = lax.axis_index("c") * nsub + lax.axis_index("s")

    pltpu.sync_copy(zin, zer)

    pltpu.sync_copy(anch, av)
    lane = lax.iota(jnp.int32, _LANES)
    r = jnp.max(plsc.load_gather(av, [lane]))
    c = jnp.max(plsc.load_gather(av, [_LANES + lane]))

    zero16 = jnp.broadcast_to(jnp.int32(0), (_LANES,))
    for ib in idxs:
        plsc.store_scatter(ib, [lane], zero16)
        plsc.store_scatter(ib, [_LANES + lane], zero16,
                           mask=(_LANES + lane) < _U)

    tconst = []
    for (k, col0, npx) in _TYPES:
        size, scale, pad = _SIZES[k], _SCALES[k], _PADS[k]
        r0 = scale * r - pad
        c0 = scale * c - pad
        chunks = []
        for j in range((npx + _LANES - 1) // _LANES):
            oc = _LANES * j + lane
            chunks.append((oc, jnp.clip(c0 + col0 + oc, 0, _H - 1),
                           oc < npx))
        nlo = jnp.clip(-c0 - col0, 0, npx)
        nhi = jnp.clip(c0 + col0 + npx - _H, 0, npx)
        tconst.append((k, col0, npx, r0, chunks, nlo, nhi))

    def unit_u(rid, t):
        return rid - _STARTS[t]

    def build_idx(slot_rid, par):
        for t, (k, col0, npx, r0, chunks, nlo, nhi) in enumerate(tconst):
            lo = _STARTS[t]
            hi = _STARTS[t + 1] if t + 1 < len(_STARTS) else _NUNITS
            u = unit_u(slot_rid, t)
            srow = r0 + u
            valid = jnp.logical_and(srow >= 0, srow < _H)
            on = jnp.logical_and(
                jnp.logical_and(slot_rid >= lo, slot_rid < hi), valid)

            @pl.when(on)
            def _():
                rbase = srow * _H
                for (oc, ccl, m) in chunks:
                    plsc.store_scatter(idxs[par], [oc], rbase + ccl, mask=m)

    def finish_unit(slot_rid, par):
        for t, (k, col0, npx, r0, chunks, nlo, nhi) in enumerate(tconst):
            lo = _STARTS[t]
            hi = _STARTS[t + 1] if t + 1 < len(_STARTS) else _NUNITS
            u = unit_u(slot_rid, t)
            srow = r0 + u
            valid = jnp.logical_and(srow >= 0, srow < _H)
            on = jnp.logical_and(slot_rid >= lo, slot_rid < hi)
            base = _SIZES[k] * u + col0
            buf = bufs[par]

            @pl.when(jnp.logical_and(on, valid))
            def _():
                def zlo(p, carry):
                    pltpu.sync_copy(zin.at[pl.ds(0, 1)],
                                    buf.at[pl.ds(p, 1)])
                    return carry

                def zhi(p, carry):
                    pltpu.sync_copy(zin.at[pl.ds(0, 1)],
                                    buf.at[pl.ds(npx - 1 - p, 1)])
                    return carry

                lax.fori_loop(0, nlo, zlo, jnp.int32(0))
                lax.fori_loop(0, nhi, zhi, jnp.int32(0))
                pltpu.async_copy(buf.at[pl.ds(0, npx)],
                                 outs[k].at[pl.ds(base, npx)], sem_o).wait()

            @pl.when(jnp.logical_and(on, jnp.logical_not(valid)))
            def _():
                off = 0
                while off < npx:
                    n = min(_LANES, npx - off)
                    pltpu.sync_copy(zer.at[pl.ds(0, n)],
                                    outs[k].at[pl.ds(base + off, n)])
                    off += n

    nslots = (_NUNITS + nworkers - 1) // nworkers
    rids = [jnp.minimum(wid + s * nworkers, _NUNITS - 1)
            for s in range(nslots)]

    build_idx(rids[0], 0)
    g = pltpu.async_copy(ffr.at[idxs[0]], bufs[0], gsems[0])
    for s in range(nslots):
        par = s % 2
        nxt = (s + 1) % 2
        g.wait()
        if s + 1 < nslots:
            build_idx(rids[s + 1], nxt)
            g = pltpu.async_copy(ffr.at[idxs[nxt]], bufs[nxt], gsems[nxt])
        finish_unit(rids[s], par)

    @pl.when(wid == nworkers - 1)
    def _():
        s25 = _HC * r + c
        pltpu.sync_copy(cor.at[pl.ds(s25, 1)], bufa.at[pl.ds(0, 1)])
        pltpu.sync_copy(bufa.at[pl.ds(0, 1)], q3)


def kernel(full_feature, corr_feature, anchor):
    B, C, H, W = full_feature.shape
    ffr = _to_rows(full_feature)
    cor = _to_rows(corr_feature)
    a32 = anchor.astype(jnp.int32)
    anch = jnp.concatenate([
        jnp.broadcast_to(a32[0], (_LANES,)),
        jnp.broadcast_to(a32[1], (_LANES,)),
    ])
    f32 = jnp.float32
    zin = jnp.zeros((_LANES, 8, 128), f32)

    mesh = plsc.VectorSubcoreMesh(core_axis_name="c", subcore_axis_name="s")
    nworkers = mesh.num_cores * mesh.num_subcores

    run = pl.kernel(
        functools.partial(_body, nsub=mesh.num_subcores, nworkers=nworkers),
        out_type=(
            jax.ShapeDtypeStruct((61 * 61, 8, 128), f32),
            jax.ShapeDtypeStruct((31 * 31, 8, 128), f32),
            jax.ShapeDtypeStruct((15 * 15, 8, 128), f32),
            jax.ShapeDtypeStruct((1, 8, 128), f32),
        ),
        mesh=mesh,
        compiler_params=pltpu.CompilerParams(use_tc_tiling_on_sc=True,
                                             needs_layout_passes=False),
        scratch_types=[
            pltpu.VMEM((2 * _LANES,), jnp.int32),
            pltpu.VMEM((_U, 8, 128), f32),
            pltpu.VMEM((_U, 8, 128), f32),
            pltpu.VMEM((_LANES, 8, 128), f32),
            pltpu.VMEM((_U,), jnp.int32),
            pltpu.VMEM((_U,), jnp.int32),
            pltpu.SemaphoreType.DMA,
            pltpu.SemaphoreType.DMA,
            pltpu.SemaphoreType.DMA,
            pltpu.SemaphoreType.DMA,
        ],
    )
    q0, q1, q2, q3 = run(ffr, cor, anch, zin)
    return (_from_rows(q0, 61, 61), _from_rows(q1, 31, 31),
            _from_rows(q2, 15, 15), _from_rows(q3, 1, 1))
